# Initial kernel scaffold; baseline (speedup 1.0000x reference)
import jax
import jax.numpy as jnp
from jax.experimental import pallas as pl


def kernel(x, edge_index, W1, b1, W2, b2, Wa, ba, Wf, bf, Wd, bd):
    raise NotImplementedError("write your pallas kernel here")


# speedup vs baseline: 15.7122x; 15.7122x over previous
import functools

import jax
import jax.numpy as jnp
from jax import lax
from jax.experimental import pallas as pl
from jax.experimental.pallas import tpu as pltpu
from jax.experimental.pallas import tpu_sc as plsc

N = 10000
E = 160000
D = 256
H = 32

NW = 32
CH = 128
NCHUNK = 40
EPT = CH * NCHUNK
EP = EPT * NW
NP = 10240
RPT = NP // 16
DUMMY = NP


def _deg_body(dst3, zeros1, out, dst_v, ones_v, acc):
    c = lax.axis_index("c")
    s = lax.axis_index("s")
    wid = c * 16 + s
    pltpu.sync_copy(dst3.at[wid], dst_v)
    for i in range(CH // 16):
        ones_v[pl.ds(16 * i, 16)] = jnp.full((16,), 1.0, jnp.float32)
    pltpu.sync_copy(zeros1.at[pl.ds(s * RPT, RPT)], acc.at[pl.ds(s * RPT, RPT)])
    plsc.subcore_barrier()

    def body(j, _):
        pltpu.sync_copy(ones_v, acc.at[dst_v.at[j]], add=True)
        return ()

    lax.fori_loop(0, NCHUNK, body, ())
    plsc.subcore_barrier()
    pltpu.sync_copy(acc.at[pl.ds(s * RPT, RPT)], out.at[c, pl.ds(s * RPT, RPT)])


def _seg_body(table, src3, dst3, zeros2, out, src_v, dst_v, rows, acc, sem):
    c = lax.axis_index("c")
    s = lax.axis_index("s")
    wid = c * 16 + s
    pltpu.sync_copy(src3.at[wid], src_v)
    pltpu.sync_copy(dst3.at[wid], dst_v)
    pltpu.sync_copy(zeros2.at[pl.ds(s * RPT, RPT)], acc.at[pl.ds(s * RPT, RPT)])
    plsc.subcore_barrier()

    def body(j, _):
        pltpu.async_copy(table.at[src_v.at[j]], rows, sem).wait()
        pltpu.sync_copy(rows, acc.at[dst_v.at[j]], add=True)
        return ()

    lax.fori_loop(0, NCHUNK, body, ())
    plsc.subcore_barrier()
    pltpu.sync_copy(acc.at[pl.ds(s * RPT, RPT)], out.at[c, pl.ds(s * RPT, RPT)])


@functools.lru_cache(maxsize=None)
def _sc_kernels():
    mesh = plsc.VectorSubcoreMesh(core_axis_name="c", subcore_axis_name="s")
    deg = pl.kernel(
        _deg_body,
        out_type=jax.ShapeDtypeStruct((2, NP), jnp.float32),
        mesh=mesh,
        scratch_types=[
            pltpu.VMEM((NCHUNK, CH), jnp.int32),
            pltpu.VMEM((CH,), jnp.float32),
            pltpu.VMEM_SHARED((NP + 8,), jnp.float32),
        ],
    )
    seg = pl.kernel(
        _seg_body,
        out_type=jax.ShapeDtypeStruct((2, NP, H), jnp.float32),
        mesh=mesh,
        compiler_params=pltpu.CompilerParams(use_tc_tiling_on_sc=False),
        scratch_types=[
            pltpu.VMEM((NCHUNK, CH), jnp.int32),
            pltpu.VMEM((NCHUNK, CH), jnp.int32),
            pltpu.VMEM((CH, H), jnp.float32),
            pltpu.VMEM_SHARED((NP + 8, H), jnp.float32),
            pltpu.SemaphoreType.DMA,
        ],
    )
    return deg, seg


def _dis(degcol_ref):
    deg = degcol_ref[:, 0:1] + degcol_ref[:, 1:2] + 1.0
    return lax.rsqrt(deg)


def _tc1_body(x_ref, w1_ref, degcol_ref, hs_ref):
    dis = _dis(degcol_ref)
    xw = jnp.dot(x_ref[...], w1_ref[...], preferred_element_type=jnp.float32)
    hs_ref[...] = xw * dis


def _tc2_body(p1a_ref, p1b_ref, hs1_ref, degcol_ref, b1_ref, w2_ref, out_ref):
    dis = _dis(degcol_ref)
    h1 = jnp.maximum(
        dis * (p1a_ref[...] + p1b_ref[...] + hs1_ref[...]) + b1_ref[...], 0.0)
    out_ref[...] = jnp.dot(h1, w2_ref[...],
                           preferred_element_type=jnp.float32) * dis


def _tc3_body(p2a_ref, p2b_ref, hs2_ref, degcol_ref, b2_ref, wa_ref, ba_ref,
              wf_ref, bf_ref, wd_ref, bd_ref, out_ref):
    dis = _dis(degcol_ref)
    h2 = jnp.maximum(
        dis * (p2a_ref[...] + p2b_ref[...] + hs2_ref[...]) + b2_ref[...], 0.0)
    za = jnp.dot(h2, wa_ref[...], preferred_element_type=jnp.float32) + ba_ref[...]
    attn = 1.0 / (1.0 + jnp.exp(-za))
    feat = jnp.dot(h2, wf_ref[...], preferred_element_type=jnp.float32) + bf_ref[...]
    pooled = jnp.sum(attn * feat, axis=0, keepdims=True)
    out_ref[...] = jnp.dot(pooled, wd_ref[...],
                           preferred_element_type=jnp.float32) + bd_ref[...]


def kernel(x, edge_index, W1, b1, W2, b2, Wa, ba, Wf, bf, Wd, bd):
    f32 = jnp.float32
    src = edge_index[0]
    dst = edge_index[1]
    src3 = jnp.concatenate(
        [src, jnp.zeros((EP - E,), jnp.int32)]).reshape(NW, NCHUNK, CH)
    dst3 = jnp.concatenate(
        [dst, jnp.full((EP - E,), DUMMY, jnp.int32)]).reshape(NW, NCHUNK, CH)
    zeros1 = jnp.zeros((NP,), f32)
    zeros2 = jnp.zeros((NP, H), f32)

    _deg_kernel, _seg_kernel = _sc_kernels()
    degp = _deg_kernel(dst3, zeros1)
    degcol = jnp.transpose(degp)[:N]

    hs1 = pl.pallas_call(
        _tc1_body,
        out_shape=jax.ShapeDtypeStruct((N, H), f32),
    )(x, W1, degcol)

    P1 = _seg_kernel(hs1, src3, dst3, zeros2)

    hs2 = pl.pallas_call(
        _tc2_body,
        out_shape=jax.ShapeDtypeStruct((N, H), f32),
    )(P1[0, :N], P1[1, :N], hs1, degcol, b1.reshape(1, H), W2)

    P2 = _seg_kernel(hs2, src3, dst3, zeros2)

    out = pl.pallas_call(
        _tc3_body,
        out_shape=jax.ShapeDtypeStruct((1, 1), f32),
    )(P2[0, :N], P2[1, :N], hs2, degcol, b2.reshape(1, H), Wa,
      ba.reshape(1, H), Wf, bf.reshape(1, H), Wd, bd.reshape(1, 1))
    return out


# speedup vs baseline: 16.9216x; 1.0770x over previous
import functools

import jax
import jax.numpy as jnp
from jax import lax
from jax.experimental import pallas as pl
from jax.experimental.pallas import tpu as pltpu
from jax.experimental.pallas import tpu_sc as plsc

N = 10000
E = 160000
D = 256
H = 32

NW = 32
CH = 128
NCHUNK = 40
EPT = CH * NCHUNK
EP = EPT * NW
NP = 10240
RPT = NP // 16
DUMMY = NP


def _deg_body(dst3, zeros1, out, dst_v, ones_v, acc):
    c = lax.axis_index("c")
    s = lax.axis_index("s")
    wid = c * 16 + s
    pltpu.sync_copy(dst3.at[wid], dst_v)
    for i in range(CH // 16):
        ones_v[pl.ds(16 * i, 16)] = jnp.full((16,), 1.0, jnp.float32)
    pltpu.sync_copy(zeros1.at[pl.ds(s * RPT, RPT)], acc.at[pl.ds(s * RPT, RPT)])
    plsc.subcore_barrier()

    def body(j, _):
        pltpu.sync_copy(ones_v, acc.at[dst_v.at[j]], add=True)
        return ()

    lax.fori_loop(0, NCHUNK, body, ())
    plsc.subcore_barrier()
    pltpu.sync_copy(acc.at[pl.ds(s * RPT, RPT)], out.at[c, pl.ds(s * RPT, RPT)])


def _seg_body(table, src3, dst3, zeros2, out, src_v, dst_v, rows0, rows1,
              acc, sem0, sem1):
    c = lax.axis_index("c")
    s = lax.axis_index("s")
    wid = c * 16 + s
    pltpu.sync_copy(src3.at[wid], src_v)
    pltpu.sync_copy(dst3.at[wid], dst_v)
    pltpu.sync_copy(zeros2.at[pl.ds(s * RPT, RPT)], acc.at[pl.ds(s * RPT, RPT)])
    plsc.subcore_barrier()

    bufs = (rows0, rows1)
    sems = (sem0, sem1)
    pltpu.async_copy(table.at[src_v.at[0]], rows0, sem0)

    def body(o, _):
        for b in range(2):
            j = 2 * o + b
            pltpu.make_async_copy(table.at[src_v.at[j]], bufs[b],
                                  sems[b]).wait()
            nxt = bufs[1 - b]

            def start_next():
                pltpu.async_copy(table.at[src_v.at[j + 1]], nxt, sems[1 - b])

            if b == 0:
                start_next()
            else:
                pl.when(o < NCHUNK // 2 - 1)(start_next)
            pltpu.sync_copy(bufs[b], acc.at[dst_v.at[j]], add=True)
        return ()

    lax.fori_loop(0, NCHUNK // 2, body, ())
    plsc.subcore_barrier()
    pltpu.sync_copy(acc.at[pl.ds(s * RPT, RPT)], out.at[c, pl.ds(s * RPT, RPT)])


@functools.lru_cache(maxsize=None)
def _sc_kernels():
    mesh = plsc.VectorSubcoreMesh(core_axis_name="c", subcore_axis_name="s")
    deg = pl.kernel(
        _deg_body,
        out_type=jax.ShapeDtypeStruct((2, NP), jnp.float32),
        mesh=mesh,
        scratch_types=[
            pltpu.VMEM((NCHUNK, CH), jnp.int32),
            pltpu.VMEM((CH,), jnp.float32),
            pltpu.VMEM_SHARED((NP + 8,), jnp.float32),
        ],
    )
    seg = pl.kernel(
        _seg_body,
        out_type=jax.ShapeDtypeStruct((2, NP, H), jnp.float32),
        mesh=mesh,
        compiler_params=pltpu.CompilerParams(use_tc_tiling_on_sc=False),
        scratch_types=[
            pltpu.VMEM((NCHUNK, CH), jnp.int32),
            pltpu.VMEM((NCHUNK, CH), jnp.int32),
            pltpu.VMEM((CH, H), jnp.float32),
            pltpu.VMEM((CH, H), jnp.float32),
            pltpu.VMEM_SHARED((NP + 8, H), jnp.float32),
            pltpu.SemaphoreType.DMA,
            pltpu.SemaphoreType.DMA,
        ],
    )
    return deg, seg


def _dis(degcol_ref):
    deg = degcol_ref[:, 0:1] + degcol_ref[:, 1:2] + 1.0
    return lax.rsqrt(deg)


def _tc1_body(x_ref, w1_ref, degcol_ref, hs_ref):
    dis = _dis(degcol_ref)
    xw = jnp.dot(x_ref[...], w1_ref[...], preferred_element_type=jnp.float32)
    hs_ref[...] = xw * dis


def _tc2_body(p1_ref, hs1_ref, degcol_ref, b1_ref, w2_ref, out_ref):
    dis = _dis(degcol_ref)
    p1 = p1_ref[0, :N] + p1_ref[1, :N]
    h1 = jnp.maximum(dis * (p1 + hs1_ref[...]) + b1_ref[...], 0.0)
    out_ref[...] = jnp.dot(h1, w2_ref[...],
                           preferred_element_type=jnp.float32) * dis


def _tc3_body(p2_ref, hs2_ref, degcol_ref, b2_ref, wa_ref, ba_ref,
              wf_ref, bf_ref, wd_ref, bd_ref, out_ref):
    dis = _dis(degcol_ref)
    p2 = p2_ref[0, :N] + p2_ref[1, :N]
    h2 = jnp.maximum(dis * (p2 + hs2_ref[...]) + b2_ref[...], 0.0)
    za = jnp.dot(h2, wa_ref[...], preferred_element_type=jnp.float32) + ba_ref[...]
    attn = 1.0 / (1.0 + jnp.exp(-za))
    feat = jnp.dot(h2, wf_ref[...], preferred_element_type=jnp.float32) + bf_ref[...]
    pooled = jnp.sum(attn * feat, axis=0, keepdims=True)
    out_ref[...] = jnp.dot(pooled, wd_ref[...],
                           preferred_element_type=jnp.float32) + bd_ref[...]


def kernel(x, edge_index, W1, b1, W2, b2, Wa, ba, Wf, bf, Wd, bd):
    f32 = jnp.float32
    src = edge_index[0]
    dst = edge_index[1]
    src3 = jnp.concatenate(
        [src, jnp.zeros((EP - E,), jnp.int32)]).reshape(NW, NCHUNK, CH)
    dst3 = jnp.concatenate(
        [dst, jnp.full((EP - E,), DUMMY, jnp.int32)]).reshape(NW, NCHUNK, CH)
    zeros1 = jnp.zeros((NP,), f32)
    zeros2 = jnp.zeros((NP, H), f32)

    _deg_kernel, _seg_kernel = _sc_kernels()
    degp = _deg_kernel(dst3, zeros1)
    degcol = jnp.transpose(degp)[:N]

    hs1 = pl.pallas_call(
        _tc1_body,
        out_shape=jax.ShapeDtypeStruct((N, H), f32),
    )(x, W1, degcol)

    P1 = _seg_kernel(hs1, src3, dst3, zeros2)

    hs2 = pl.pallas_call(
        _tc2_body,
        out_shape=jax.ShapeDtypeStruct((N, H), f32),
    )(P1, hs1, degcol, b1.reshape(1, H), W2)

    P2 = _seg_kernel(hs2, src3, dst3, zeros2)

    out = pl.pallas_call(
        _tc3_body,
        out_shape=jax.ShapeDtypeStruct((1, 1), f32),
    )(P2, hs2, degcol, b2.reshape(1, H), Wa,
      ba.reshape(1, H), Wf, bf.reshape(1, H), Wd, bd.reshape(1, 1))
    return out


# speedup vs baseline: 19.0558x; 1.1261x over previous
import functools

import jax
import jax.numpy as jnp
from jax import lax
from jax.experimental import pallas as pl
from jax.experimental.pallas import tpu as pltpu
from jax.experimental.pallas import tpu_sc as plsc

N = 10000
E = 160000
D = 256
H = 32

NW = 32
CH = 128
NCHUNK = 40
EPT = CH * NCHUNK
EP = EPT * NW
NP = 10240
RPT = NP // 16
DUMMY = NP


def _deg_body(dst3, zeros1, out, dst_v, ones_v, acc):
    c = lax.axis_index("c")
    s = lax.axis_index("s")
    wid = c * 16 + s
    pltpu.sync_copy(dst3.at[wid], dst_v)
    for i in range(CH // 16):
        ones_v[pl.ds(16 * i, 16)] = jnp.full((16,), 1.0, jnp.float32)
    pltpu.sync_copy(zeros1.at[pl.ds(s * RPT, RPT)], acc.at[pl.ds(s * RPT, RPT)])
    plsc.subcore_barrier()

    def body(j, _):
        pltpu.sync_copy(ones_v, acc.at[dst_v.at[j]], add=True)
        return ()

    lax.fori_loop(0, NCHUNK, body, ())
    plsc.subcore_barrier()
    pltpu.sync_copy(acc.at[pl.ds(s * RPT, RPT)], out.at[c, pl.ds(s * RPT, RPT)])


def _seg_body(table, src3, dst3, zeros2, out, src_v, dst_v,
              rows0, rows1, rows2, rows3,
              acc, g0, g1, g2, g3, s0, s1, s2, s3):
    c = lax.axis_index("c")
    s = lax.axis_index("s")
    wid = c * 16 + s
    pltpu.sync_copy(src3.at[wid], src_v)
    pltpu.sync_copy(dst3.at[wid], dst_v)
    pltpu.sync_copy(zeros2.at[pl.ds(s * RPT, RPT)], acc.at[pl.ds(s * RPT, RPT)])
    plsc.subcore_barrier()

    bufs = (rows0, rows1, rows2, rows3)
    gsem = (g0, g1, g2, g3)
    ssem = (s0, s1, s2, s3)
    NB = 4
    NO = NCHUNK // NB

    pltpu.async_copy(table.at[src_v.at[0]], bufs[0], gsem[0])
    pltpu.async_copy(table.at[src_v.at[1]], bufs[1], gsem[1])

    def body(o, _):
        for b in range(NB):
            j = NB * o + b
            bb = (b + 2) % NB
            pltpu.make_async_copy(table.at[src_v.at[j]], bufs[b],
                                  gsem[b]).wait()
            pltpu.async_copy(bufs[b], acc.at[dst_v.at[j]], ssem[b], add=True)

            def wait_sc():
                pltpu.make_async_copy(bufs[bb], acc.at[dst_v.at[j]],
                                      ssem[bb]).wait()

            def start_g():
                pltpu.async_copy(table.at[src_v.at[j + 2]], bufs[bb],
                                 gsem[bb])

            if b < 2:
                pl.when(o > 0)(wait_sc)
                start_g()
            else:
                wait_sc()
                pl.when(o < NO - 1)(start_g)
        return ()

    lax.fori_loop(0, NO, body, ())
    pltpu.make_async_copy(bufs[2], acc.at[dst_v.at[NCHUNK - 2]],
                          ssem[2]).wait()
    pltpu.make_async_copy(bufs[3], acc.at[dst_v.at[NCHUNK - 1]],
                          ssem[3]).wait()
    plsc.subcore_barrier()
    pltpu.sync_copy(acc.at[pl.ds(s * RPT, RPT)], out.at[c, pl.ds(s * RPT, RPT)])


@functools.lru_cache(maxsize=None)
def _sc_kernels():
    mesh = plsc.VectorSubcoreMesh(core_axis_name="c", subcore_axis_name="s")
    deg = pl.kernel(
        _deg_body,
        out_type=jax.ShapeDtypeStruct((2, NP), jnp.float32),
        mesh=mesh,
        scratch_types=[
            pltpu.VMEM((NCHUNK, CH), jnp.int32),
            pltpu.VMEM((CH,), jnp.float32),
            pltpu.VMEM_SHARED((NP + 8,), jnp.float32),
        ],
    )
    seg = pl.kernel(
        _seg_body,
        out_type=jax.ShapeDtypeStruct((2, NP, H), jnp.float32),
        mesh=mesh,
        compiler_params=pltpu.CompilerParams(use_tc_tiling_on_sc=False),
        scratch_types=[
            pltpu.VMEM((NCHUNK, CH), jnp.int32),
            pltpu.VMEM((NCHUNK, CH), jnp.int32),
            pltpu.VMEM((CH, H), jnp.float32),
            pltpu.VMEM((CH, H), jnp.float32),
            pltpu.VMEM((CH, H), jnp.float32),
            pltpu.VMEM((CH, H), jnp.float32),
            pltpu.VMEM_SHARED((NP + 8, H), jnp.float32),
        ] + [pltpu.SemaphoreType.DMA] * 8,
    )
    return deg, seg


def _dis(degcol_ref):
    deg = degcol_ref[:, 0:1] + degcol_ref[:, 1:2] + 1.0
    return lax.rsqrt(deg)


def _tc1_body(x_ref, w1_ref, degcol_ref, hs_ref):
    dis = _dis(degcol_ref)
    xw = jnp.dot(x_ref[...], w1_ref[...], preferred_element_type=jnp.float32)
    hs_ref[...] = xw * dis


def _tc2_body(p1_ref, hs1_ref, degcol_ref, b1_ref, w2_ref, out_ref):
    dis = _dis(degcol_ref)
    p1 = p1_ref[0, :N] + p1_ref[1, :N]
    h1 = jnp.maximum(dis * (p1 + hs1_ref[...]) + b1_ref[...], 0.0)
    out_ref[...] = jnp.dot(h1, w2_ref[...],
                           preferred_element_type=jnp.float32) * dis


def _tc3_body(p2_ref, hs2_ref, degcol_ref, b2_ref, wa_ref, ba_ref,
              wf_ref, bf_ref, wd_ref, bd_ref, out_ref):
    dis = _dis(degcol_ref)
    p2 = p2_ref[0, :N] + p2_ref[1, :N]
    h2 = jnp.maximum(dis * (p2 + hs2_ref[...]) + b2_ref[...], 0.0)
    za = jnp.dot(h2, wa_ref[...], preferred_element_type=jnp.float32) + ba_ref[...]
    attn = 1.0 / (1.0 + jnp.exp(-za))
    feat = jnp.dot(h2, wf_ref[...], preferred_element_type=jnp.float32) + bf_ref[...]
    pooled = jnp.sum(attn * feat, axis=0, keepdims=True)
    out_ref[...] = jnp.dot(pooled, wd_ref[...],
                           preferred_element_type=jnp.float32) + bd_ref[...]


def kernel(x, edge_index, W1, b1, W2, b2, Wa, ba, Wf, bf, Wd, bd):
    f32 = jnp.float32
    src = edge_index[0]
    dst = edge_index[1]
    src3 = jnp.concatenate(
        [src, jnp.zeros((EP - E,), jnp.int32)]).reshape(NW, NCHUNK, CH)
    dst3 = jnp.concatenate(
        [dst, jnp.full((EP - E,), DUMMY, jnp.int32)]).reshape(NW, NCHUNK, CH)
    zeros1 = jnp.zeros((NP,), f32)
    zeros2 = jnp.zeros((NP, H), f32)

    _deg_kernel, _seg_kernel = _sc_kernels()
    degp = _deg_kernel(dst3, zeros1)
    degcol = jnp.transpose(degp)[:N]

    hs1 = pl.pallas_call(
        _tc1_body,
        out_shape=jax.ShapeDtypeStruct((N, H), f32),
    )(x, W1, degcol)

    P1 = _seg_kernel(hs1, src3, dst3, zeros2)

    hs2 = pl.pallas_call(
        _tc2_body,
        out_shape=jax.ShapeDtypeStruct((N, H), f32),
    )(P1, hs1, degcol, b1.reshape(1, H), W2)

    P2 = _seg_kernel(hs2, src3, dst3, zeros2)

    out = pl.pallas_call(
        _tc3_body,
        out_shape=jax.ShapeDtypeStruct((1, 1), f32),
    )(P2, hs2, degcol, b2.reshape(1, H), Wa,
      ba.reshape(1, H), Wf, bf.reshape(1, H), Wd, bd.reshape(1, 1))
    return out


# speedup vs baseline: 28.9198x; 1.5176x over previous
import functools

import jax
import jax.numpy as jnp
from jax import lax
from jax.experimental import pallas as pl
from jax.experimental.pallas import tpu as pltpu
from jax.experimental.pallas import tpu_sc as plsc

N = 10000
E = 160000
D = 256
H = 32

NW = 32
CH = 128
NCHUNK = 40
EPT = CH * NCHUNK
EP = EPT * NW
NP = 10240
RPT = NP // 16
DUMMY = NP


def _deg_body(dst3, zeros1, out, dst_v, ones_v, acc):
    c = lax.axis_index("c")
    s = lax.axis_index("s")
    wid = c * 16 + s
    pltpu.sync_copy(dst3.at[wid], dst_v)
    for i in range(CH // 16):
        ones_v[pl.ds(16 * i, 16)] = jnp.full((16,), 1.0, jnp.float32)
    pltpu.sync_copy(zeros1.at[pl.ds(s * RPT, RPT)], acc.at[pl.ds(s * RPT, RPT)])
    plsc.subcore_barrier()

    def body(j, _):
        pltpu.sync_copy(ones_v, acc.at[dst_v.at[j]], add=True)
        return ()

    lax.fori_loop(0, NCHUNK, body, ())
    plsc.subcore_barrier()
    pltpu.sync_copy(acc.at[pl.ds(s * RPT, RPT)], out.at[c, pl.ds(s * RPT, RPT)])


def _seg_body(table, src3, dst3, zeros2, out, src_v, dst_v,
              rows0, rows1, rows2, rows3,
              acc, tab_sh, g0, g1, g2, g3, s0, s1, s2, s3):
    c = lax.axis_index("c")
    s = lax.axis_index("s")
    wid = c * 16 + s
    pltpu.sync_copy(src3.at[wid], src_v)
    pltpu.sync_copy(dst3.at[wid], dst_v)
    pltpu.sync_copy(zeros2.at[pl.ds(s * RPT, RPT)], acc.at[pl.ds(s * RPT, RPT)])
    pltpu.sync_copy(table.at[pl.ds(s * RPT, RPT)], tab_sh.at[pl.ds(s * RPT, RPT)])
    plsc.subcore_barrier()

    bufs = (rows0, rows1, rows2, rows3)
    gsem = (g0, g1, g2, g3)
    ssem = (s0, s1, s2, s3)
    NB = 4
    NO = NCHUNK // NB

    pltpu.async_copy(tab_sh.at[src_v.at[0]], bufs[0], gsem[0])
    pltpu.async_copy(tab_sh.at[src_v.at[1]], bufs[1], gsem[1])

    def body(o, _):
        for b in range(NB):
            j = NB * o + b
            bb = (b + 2) % NB
            pltpu.make_async_copy(tab_sh.at[src_v.at[j]], bufs[b],
                                  gsem[b]).wait()
            pltpu.async_copy(bufs[b], acc.at[dst_v.at[j]], ssem[b], add=True)

            def wait_sc():
                pltpu.make_async_copy(bufs[bb], acc.at[dst_v.at[j]],
                                      ssem[bb]).wait()

            def start_g():
                pltpu.async_copy(tab_sh.at[src_v.at[j + 2]], bufs[bb],
                                 gsem[bb])

            if b < 2:
                pl.when(o > 0)(wait_sc)
                start_g()
            else:
                wait_sc()
                pl.when(o < NO - 1)(start_g)
        return ()

    lax.fori_loop(0, NO, body, ())
    pltpu.make_async_copy(bufs[2], acc.at[dst_v.at[NCHUNK - 2]],
                          ssem[2]).wait()
    pltpu.make_async_copy(bufs[3], acc.at[dst_v.at[NCHUNK - 1]],
                          ssem[3]).wait()
    plsc.subcore_barrier()
    pltpu.sync_copy(acc.at[pl.ds(s * RPT, RPT)], out.at[c, pl.ds(s * RPT, RPT)])


@functools.lru_cache(maxsize=None)
def _sc_kernels():
    mesh = plsc.VectorSubcoreMesh(core_axis_name="c", subcore_axis_name="s")
    deg = pl.kernel(
        _deg_body,
        out_type=jax.ShapeDtypeStruct((2, NP), jnp.float32),
        mesh=mesh,
        scratch_types=[
            pltpu.VMEM((NCHUNK, CH), jnp.int32),
            pltpu.VMEM((CH,), jnp.float32),
            pltpu.VMEM_SHARED((NP + 8,), jnp.float32),
        ],
    )
    seg = pl.kernel(
        _seg_body,
        out_type=jax.ShapeDtypeStruct((2, NP, H), jnp.float32),
        mesh=mesh,
        compiler_params=pltpu.CompilerParams(use_tc_tiling_on_sc=False),
        scratch_types=[
            pltpu.VMEM((NCHUNK, CH), jnp.int32),
            pltpu.VMEM((NCHUNK, CH), jnp.int32),
            pltpu.VMEM((CH, H), jnp.float32),
            pltpu.VMEM((CH, H), jnp.float32),
            pltpu.VMEM((CH, H), jnp.float32),
            pltpu.VMEM((CH, H), jnp.float32),
            pltpu.VMEM_SHARED((NP + 8, H), jnp.float32),
            pltpu.VMEM_SHARED((NP, H), jnp.float32),
        ] + [pltpu.SemaphoreType.DMA] * 8,
    )
    return deg, seg


def _dis(degcol_ref):
    deg = degcol_ref[:, 0:1] + degcol_ref[:, 1:2] + 1.0
    return lax.rsqrt(deg)


def _tc1_body(x_ref, w1_ref, degcol_ref, hs_ref):
    dis = _dis(degcol_ref)
    xw = jnp.dot(x_ref[...], w1_ref[...], preferred_element_type=jnp.float32)
    hs_ref[:N] = xw * dis
    hs_ref[N:] = jnp.zeros((NP - N, H), jnp.float32)


def _tc2_body(p1_ref, hs1_ref, degcol_ref, b1_ref, w2_ref, out_ref):
    dis = _dis(degcol_ref)
    p1 = p1_ref[0, :N] + p1_ref[1, :N]
    h1 = jnp.maximum(dis * (p1 + hs1_ref[:N]) + b1_ref[...], 0.0)
    out_ref[:N] = jnp.dot(h1, w2_ref[...],
                          preferred_element_type=jnp.float32) * dis
    out_ref[N:] = jnp.zeros((NP - N, H), jnp.float32)


def _tc3_body(p2_ref, hs2_ref, degcol_ref, b2_ref, wa_ref, ba_ref,
              wf_ref, bf_ref, wd_ref, bd_ref, out_ref):
    dis = _dis(degcol_ref)
    p2 = p2_ref[0, :N] + p2_ref[1, :N]
    h2 = jnp.maximum(dis * (p2 + hs2_ref[:N]) + b2_ref[...], 0.0)
    za = jnp.dot(h2, wa_ref[...], preferred_element_type=jnp.float32) + ba_ref[...]
    attn = 1.0 / (1.0 + jnp.exp(-za))
    feat = jnp.dot(h2, wf_ref[...], preferred_element_type=jnp.float32) + bf_ref[...]
    pooled = jnp.sum(attn * feat, axis=0, keepdims=True)
    out_ref[...] = jnp.dot(pooled, wd_ref[...],
                           preferred_element_type=jnp.float32) + bd_ref[...]


def kernel(x, edge_index, W1, b1, W2, b2, Wa, ba, Wf, bf, Wd, bd):
    f32 = jnp.float32
    src = edge_index[0]
    dst = edge_index[1]
    src3 = jnp.concatenate(
        [src, jnp.zeros((EP - E,), jnp.int32)]).reshape(NW, NCHUNK, CH)
    dst3 = jnp.concatenate(
        [dst, jnp.full((EP - E,), DUMMY, jnp.int32)]).reshape(NW, NCHUNK, CH)
    zeros1 = jnp.zeros((NP,), f32)
    zeros2 = jnp.zeros((NP, H), f32)

    _deg_kernel, _seg_kernel = _sc_kernels()
    degp = _deg_kernel(dst3, zeros1)
    degcol = jnp.transpose(degp)[:N]

    hs1 = pl.pallas_call(
        _tc1_body,
        out_shape=jax.ShapeDtypeStruct((NP, H), f32),
    )(x, W1, degcol)

    P1 = _seg_kernel(hs1, src3, dst3, zeros2)

    hs2 = pl.pallas_call(
        _tc2_body,
        out_shape=jax.ShapeDtypeStruct((NP, H), f32),
    )(P1, hs1, degcol, b1.reshape(1, H), W2)

    P2 = _seg_kernel(hs2, src3, dst3, zeros2)

    out = pl.pallas_call(
        _tc3_body,
        out_shape=jax.ShapeDtypeStruct((1, 1), f32),
    )(P2, hs2, degcol, b2.reshape(1, H), Wa,
      ba.reshape(1, H), Wf, bf.reshape(1, H), Wd, bd.reshape(1, 1))
    return out


# speedup vs baseline: 32.4702x; 1.1228x over previous
import functools

import jax
import jax.numpy as jnp
from jax import lax
from jax.experimental import pallas as pl
from jax.experimental.pallas import tpu as pltpu
from jax.experimental.pallas import tpu_sc as plsc

N = 10000
E = 160000
D = 256
H = 32

NW = 32
CH = 128
NCHUNK = 40
EPT = CH * NCHUNK
EP = EPT * NW
NP = 10240
RPT = NP // 16
DUMMY = NP


def _deg_body(ei3, zeros1, out, dst_v, ones_v, acc):
    c = lax.axis_index("c")
    s = lax.axis_index("s")
    wid = c * 16 + s
    pltpu.sync_copy(ei3.at[1, wid], dst_v)
    for i in range(CH // 16):
        ones_v[pl.ds(16 * i, 16)] = jnp.full((16,), 1.0, jnp.float32)
    pltpu.sync_copy(zeros1.at[pl.ds(s * RPT, RPT)], acc.at[pl.ds(s * RPT, RPT)])
    plsc.subcore_barrier()

    def body(j, _):
        pltpu.sync_copy(ones_v, acc.at[dst_v.at[j]], add=True)
        return ()

    lax.fori_loop(0, NCHUNK, body, ())
    plsc.subcore_barrier()
    pltpu.sync_copy(acc.at[pl.ds(s * RPT, RPT)], out.at[c, pl.ds(s * RPT, RPT)])


def _seg_body(table, ei3, zeros2, out, src_v, dst_v,
              rows0, rows1, rows2, rows3,
              acc, tab_sh, g0, g1, g2, g3, s0, s1, s2, s3):
    c = lax.axis_index("c")
    s = lax.axis_index("s")
    wid = c * 16 + s
    st0 = pltpu.async_copy(ei3.at[0, wid], src_v, g0)
    st1 = pltpu.async_copy(ei3.at[1, wid], dst_v, g1)
    st2 = pltpu.async_copy(zeros2.at[pl.ds(s * RPT, RPT)],
                           acc.at[pl.ds(s * RPT, RPT)], g2)
    st3 = pltpu.async_copy(table.at[pl.ds(s * RPT, RPT)],
                           tab_sh.at[pl.ds(s * RPT, RPT)], g3)
    st0.wait()
    st1.wait()
    st2.wait()
    st3.wait()
    plsc.subcore_barrier()

    bufs = (rows0, rows1, rows2, rows3)
    gsem = (g0, g1, g2, g3)
    ssem = (s0, s1, s2, s3)
    NB = 4
    NO = NCHUNK // NB

    pltpu.async_copy(tab_sh.at[src_v.at[0]], bufs[0], gsem[0])
    pltpu.async_copy(tab_sh.at[src_v.at[1]], bufs[1], gsem[1])

    def body(o, _):
        for b in range(NB):
            j = NB * o + b
            bb = (b + 2) % NB
            pltpu.make_async_copy(tab_sh.at[src_v.at[j]], bufs[b],
                                  gsem[b]).wait()
            pltpu.async_copy(bufs[b], acc.at[dst_v.at[j]], ssem[b], add=True)

            def wait_sc():
                pltpu.make_async_copy(bufs[bb], acc.at[dst_v.at[j]],
                                      ssem[bb]).wait()

            def start_g():
                pltpu.async_copy(tab_sh.at[src_v.at[j + 2]], bufs[bb],
                                 gsem[bb])

            if b < 2:
                pl.when(o > 0)(wait_sc)
                start_g()
            else:
                wait_sc()
                pl.when(o < NO - 1)(start_g)
        return ()

    lax.fori_loop(0, NO, body, ())
    pltpu.make_async_copy(bufs[2], acc.at[dst_v.at[NCHUNK - 2]],
                          ssem[2]).wait()
    pltpu.make_async_copy(bufs[3], acc.at[dst_v.at[NCHUNK - 1]],
                          ssem[3]).wait()
    plsc.subcore_barrier()
    pltpu.sync_copy(acc.at[pl.ds(s * RPT, RPT)], out.at[c, pl.ds(s * RPT, RPT)])


@functools.lru_cache(maxsize=None)
def _sc_kernels():
    mesh = plsc.VectorSubcoreMesh(core_axis_name="c", subcore_axis_name="s")
    deg = pl.kernel(
        _deg_body,
        out_type=jax.ShapeDtypeStruct((2, NP), jnp.float32),
        mesh=mesh,
        scratch_types=[
            pltpu.VMEM((NCHUNK, CH), jnp.int32),
            pltpu.VMEM((CH,), jnp.float32),
            pltpu.VMEM_SHARED((NP + 8,), jnp.float32),
        ],
    )
    seg = pl.kernel(
        _seg_body,
        out_type=jax.ShapeDtypeStruct((2, NP, H), jnp.float32),
        mesh=mesh,
        compiler_params=pltpu.CompilerParams(use_tc_tiling_on_sc=False),
        scratch_types=[
            pltpu.VMEM((NCHUNK, CH), jnp.int32),
            pltpu.VMEM((NCHUNK, CH), jnp.int32),
            pltpu.VMEM((CH, H), jnp.float32),
            pltpu.VMEM((CH, H), jnp.float32),
            pltpu.VMEM((CH, H), jnp.float32),
            pltpu.VMEM((CH, H), jnp.float32),
            pltpu.VMEM_SHARED((NP + 8, H), jnp.float32),
            pltpu.VMEM_SHARED((NP + 8, H), jnp.float32),
        ] + [pltpu.SemaphoreType.DMA] * 8,
    )
    return deg, seg


def _tc1_body(x_ref, w1_ref, degp_ref, hs_ref, disb_ref):
    deg = degp_ref[0:1, :] + degp_ref[1:2, :] + 1.0
    dis_col = jnp.transpose(lax.rsqrt(deg))
    disb = jnp.broadcast_to(dis_col, (NP, H))
    disb_ref[...] = disb
    xw = jnp.dot(x_ref[...], w1_ref[...], preferred_element_type=jnp.float32)
    hs_ref[:N] = xw * disb[:N]
    hs_ref[N:] = jnp.zeros((NP - N, H), jnp.float32)


def _tc2_body(p1_ref, hs1_ref, disb_ref, b1_ref, w2_ref, out_ref):
    disb = disb_ref[:N]
    p1 = p1_ref[0, :N] + p1_ref[1, :N]
    h1 = jnp.maximum(disb * (p1 + hs1_ref[:N]) + b1_ref[...], 0.0)
    out_ref[:N] = jnp.dot(h1, w2_ref[...],
                          preferred_element_type=jnp.float32) * disb
    out_ref[N:] = jnp.zeros((NP - N, H), jnp.float32)


def _tc3_body(p2_ref, hs2_ref, disb_ref, b2_ref, wa_ref, ba_ref,
              wf_ref, bf_ref, wd_ref, bd_ref, out_ref):
    disb = disb_ref[:N]
    p2 = p2_ref[0, :N] + p2_ref[1, :N]
    h2 = jnp.maximum(disb * (p2 + hs2_ref[:N]) + b2_ref[...], 0.0)
    za = jnp.dot(h2, wa_ref[...], preferred_element_type=jnp.float32) + ba_ref[...]
    attn = 1.0 / (1.0 + jnp.exp(-za))
    feat = jnp.dot(h2, wf_ref[...], preferred_element_type=jnp.float32) + bf_ref[...]
    pooled = jnp.sum(attn * feat, axis=0, keepdims=True)
    out_ref[...] = jnp.dot(pooled, wd_ref[...],
                           preferred_element_type=jnp.float32) + bd_ref[...]


def kernel(x, edge_index, W1, b1, W2, b2, Wa, ba, Wf, bf, Wd, bd):
    f32 = jnp.float32
    ei3 = jnp.pad(edge_index, ((0, 0), (0, EP - E)),
                  constant_values=DUMMY).reshape(2, NW, NCHUNK, CH)
    zeros1 = jnp.zeros((NP,), f32)
    zeros2 = jnp.zeros((NP, H), f32)

    _deg_kernel, _seg_kernel = _sc_kernels()
    degp = _deg_kernel(ei3, zeros1)

    hs1, disb = pl.pallas_call(
        _tc1_body,
        out_shape=[jax.ShapeDtypeStruct((NP, H), f32),
                   jax.ShapeDtypeStruct((NP, H), f32)],
    )(x, W1, degp)

    P1 = _seg_kernel(hs1, ei3, zeros2)

    hs2 = pl.pallas_call(
        _tc2_body,
        out_shape=jax.ShapeDtypeStruct((NP, H), f32),
    )(P1, hs1, disb, b1.reshape(1, H), W2)

    P2 = _seg_kernel(hs2, ei3, zeros2)

    out = pl.pallas_call(
        _tc3_body,
        out_shape=jax.ShapeDtypeStruct((1, 1), f32),
    )(P2, hs2, disb, b2.reshape(1, H), Wa,
      ba.reshape(1, H), Wf, bf.reshape(1, H), Wd, bd.reshape(1, 1))
    return out


# speedup vs baseline: 38.5223x; 1.1864x over previous
import functools

import jax
import jax.numpy as jnp
from jax import lax
from jax.experimental import pallas as pl
from jax.experimental.pallas import tpu as pltpu
from jax.experimental.pallas import tpu_sc as plsc

N = 10000
E = 160000
D = 256
H = 32

NW = 32
CH = 128
NCHUNK = 40
EPT = CH * NCHUNK
EP = EPT * NW
NP = 10240
RPT = NP // 16
DUMMY = NP


def _deg_body(ei3, zeros1, out, dst_v, ones_v, acc):
    c = lax.axis_index("c")
    s = lax.axis_index("s")
    wid = c * 16 + s
    pltpu.sync_copy(ei3.at[1, wid], dst_v)
    for i in range(CH // 16):
        ones_v[pl.ds(16 * i, 16)] = jnp.full((16,), 1.0, jnp.float32)
    pltpu.sync_copy(zeros1.at[pl.ds(s * RPT, RPT)], acc.at[pl.ds(s * RPT, RPT)])
    plsc.subcore_barrier()

    def body(j, _):
        pltpu.sync_copy(ones_v, acc.at[dst_v.at[j]], add=True)
        return ()

    lax.fori_loop(0, NCHUNK, body, ())
    plsc.subcore_barrier()
    pltpu.sync_copy(acc.at[pl.ds(s * RPT, RPT)], out.at[c, pl.ds(s * RPT, RPT)])


def _seg_body(table, ei3, zeros2, out, src_v, dst_v,
              rows0, rows1, rows2, rows3,
              acc, tab_sh, g0, g1, g2, g3, s0, s1, s2, s3):
    c = lax.axis_index("c")
    s = lax.axis_index("s")
    wid = c * 16 + s
    st0 = pltpu.async_copy(ei3.at[0, wid], src_v, g0)
    st1 = pltpu.async_copy(ei3.at[1, wid], dst_v, g1)
    st2 = pltpu.async_copy(zeros2.at[pl.ds(s * RPT, RPT)],
                           acc.at[pl.ds(s * RPT, RPT)], g2)
    st3 = pltpu.async_copy(table.at[pl.ds(s * RPT, RPT), pl.ds(0, H)],
                           tab_sh.at[pl.ds(s * RPT, RPT)], g3)
    st0.wait()
    st1.wait()
    st2.wait()
    st3.wait()
    plsc.subcore_barrier()

    bufs = (rows0, rows1, rows2, rows3)
    gsem = (g0, g1, g2, g3)
    ssem = (s0, s1, s2, s3)
    NB = 4
    NO = NCHUNK // NB

    pltpu.async_copy(tab_sh.at[src_v.at[0]], bufs[0], gsem[0])
    pltpu.async_copy(tab_sh.at[src_v.at[1]], bufs[1], gsem[1])

    def body(o, _):
        for b in range(NB):
            j = NB * o + b
            bb = (b + 2) % NB
            pltpu.make_async_copy(tab_sh.at[src_v.at[j]], bufs[b],
                                  gsem[b]).wait()
            pltpu.async_copy(bufs[b], acc.at[dst_v.at[j]], ssem[b], add=True)

            def wait_sc():
                pltpu.make_async_copy(bufs[bb], acc.at[dst_v.at[j]],
                                      ssem[bb]).wait()

            def start_g():
                pltpu.async_copy(tab_sh.at[src_v.at[j + 2]], bufs[bb],
                                 gsem[bb])

            if b < 2:
                pl.when(o > 0)(wait_sc)
                start_g()
            else:
                wait_sc()
                pl.when(o < NO - 1)(start_g)
        return ()

    lax.fori_loop(0, NO, body, ())
    pltpu.make_async_copy(bufs[2], acc.at[dst_v.at[NCHUNK - 2]],
                          ssem[2]).wait()
    pltpu.make_async_copy(bufs[3], acc.at[dst_v.at[NCHUNK - 1]],
                          ssem[3]).wait()
    plsc.subcore_barrier()
    pltpu.sync_copy(acc.at[pl.ds(s * RPT, RPT)],
                    out.at[c, pl.ds(s * RPT, RPT), pl.ds(0, H)])


@functools.lru_cache(maxsize=None)
def _sc_kernels():
    mesh = plsc.VectorSubcoreMesh(core_axis_name="c", subcore_axis_name="s")
    deg = pl.kernel(
        _deg_body,
        out_type=jax.ShapeDtypeStruct((2, NP), jnp.float32),
        mesh=mesh,
        scratch_types=[
            pltpu.VMEM((NCHUNK, CH), jnp.int32),
            pltpu.VMEM((CH,), jnp.float32),
            pltpu.VMEM_SHARED((NP + 8,), jnp.float32),
        ],
    )
    seg = pl.kernel(
        _seg_body,
        out_type=jax.ShapeDtypeStruct((2, NP, 128), jnp.float32),
        mesh=mesh,
        compiler_params=pltpu.CompilerParams(use_tc_tiling_on_sc=False),
        scratch_types=[
            pltpu.VMEM((NCHUNK, CH), jnp.int32),
            pltpu.VMEM((NCHUNK, CH), jnp.int32),
            pltpu.VMEM((CH, H), jnp.float32),
            pltpu.VMEM((CH, H), jnp.float32),
            pltpu.VMEM((CH, H), jnp.float32),
            pltpu.VMEM((CH, H), jnp.float32),
            pltpu.VMEM_SHARED((NP + 8, H), jnp.float32),
            pltpu.VMEM_SHARED((NP + 8, H), jnp.float32),
        ] + [pltpu.SemaphoreType.DMA] * 8,
    )
    return deg, seg


def _mask32(v):
    lanes = lax.broadcasted_iota(jnp.int32, v.shape, v.ndim - 1)
    return jnp.where(lanes < H, v, 0.0)


def _tc1_body(x_ref, w1p_ref, degp_ref, hs_ref, disb_ref):
    deg = degp_ref[0:1, :] + degp_ref[1:2, :] + 1.0
    dis_col = jnp.transpose(lax.rsqrt(deg))
    disb_ref[...] = jnp.broadcast_to(dis_col, (NP, H))
    dis128 = jnp.broadcast_to(dis_col[:N], (N, 128))
    xw = jnp.dot(x_ref[...], w1p_ref[...], preferred_element_type=jnp.float32)
    hs_ref[:N] = xw * dis128
    hs_ref[N:] = jnp.zeros((NP - N, 128), jnp.float32)


def _tc2_body(p1_ref, hs1_ref, disb_ref, b1p_ref, w2p_ref, out_ref):
    dis128 = jnp.broadcast_to(disb_ref[:N, 0:1], (N, 128))
    p1 = _mask32(p1_ref[0, :N] + p1_ref[1, :N])
    h1 = jnp.maximum(dis128 * (p1 + hs1_ref[:N]) + b1p_ref[...], 0.0)
    out_ref[:N] = jnp.dot(h1, w2p_ref[...],
                          preferred_element_type=jnp.float32) * dis128
    out_ref[N:] = jnp.zeros((NP - N, 128), jnp.float32)


def _tc3_body(p2_ref, hs2_ref, disb_ref, b2p_ref, wap_ref, bap_ref,
              wfp_ref, bfp_ref, wdp_ref, bd_ref, out_ref):
    dis128 = jnp.broadcast_to(disb_ref[:N, 0:1], (N, 128))
    p2 = _mask32(p2_ref[0, :N] + p2_ref[1, :N])
    h2 = jnp.maximum(dis128 * (p2 + hs2_ref[:N]) + b2p_ref[...], 0.0)
    za = jnp.dot(h2, wap_ref[...], preferred_element_type=jnp.float32) + bap_ref[...]
    attn = 1.0 / (1.0 + jnp.exp(-za))
    feat = jnp.dot(h2, wfp_ref[...], preferred_element_type=jnp.float32) + bfp_ref[...]
    pooled = jnp.sum(attn * feat, axis=0, keepdims=True)
    out_ref[...] = jnp.dot(pooled, wdp_ref[...],
                           preferred_element_type=jnp.float32) + bd_ref[...]


def kernel(x, edge_index, W1, b1, W2, b2, Wa, ba, Wf, bf, Wd, bd):
    f32 = jnp.float32
    ei3 = jnp.pad(edge_index, ((0, 0), (0, EP - E)),
                  constant_values=DUMMY).reshape(2, NW, NCHUNK, CH)
    zeros1 = jnp.zeros((NP,), f32)
    zeros2 = jnp.zeros((NP, H), f32)

    W1p = jnp.pad(W1, ((0, 0), (0, 128 - H)))
    W2p = jnp.pad(W2, ((0, 128 - H), (0, 128 - H)))
    Wap = jnp.pad(Wa, ((0, 128 - H), (0, 128 - H)))
    Wfp = jnp.pad(Wf, ((0, 128 - H), (0, 128 - H)))
    Wdp = jnp.pad(Wd, ((0, 128 - H), (0, 0)))
    b1p = jnp.pad(b1, (0, 128 - H)).reshape(1, 128)
    b2p = jnp.pad(b2, (0, 128 - H)).reshape(1, 128)
    bap = jnp.pad(ba, (0, 128 - H)).reshape(1, 128)
    bfp = jnp.pad(bf, (0, 128 - H)).reshape(1, 128)

    _deg_kernel, _seg_kernel = _sc_kernels()
    degp = _deg_kernel(ei3, zeros1)

    hs1, disb = pl.pallas_call(
        _tc1_body,
        out_shape=[jax.ShapeDtypeStruct((NP, 128), f32),
                   jax.ShapeDtypeStruct((NP, H), f32)],
    )(x, W1p, degp)

    P1 = _seg_kernel(hs1, ei3, zeros2)

    hs2 = pl.pallas_call(
        _tc2_body,
        out_shape=jax.ShapeDtypeStruct((NP, 128), f32),
    )(P1, hs1, disb, b1p, W2p)

    P2 = _seg_kernel(hs2, ei3, zeros2)

    out = pl.pallas_call(
        _tc3_body,
        out_shape=jax.ShapeDtypeStruct((1, 1), f32),
    )(P2, hs2, disb, b2p, Wap, bap, Wfp, bfp, Wdp, bd.reshape(1, 1))
    return out
